# baseline (speedup 1.0000x reference)
import jax
import jax.numpy as jnp
from jax import lax
from jax.experimental import pallas as pl
from jax.experimental.pallas import tpu as pltpu
from jax.experimental.pallas import tpu_sc as plsc

N_NODES = 10000
N_EDGES = 320000
IN_FEAT = 128
OUT_FEAT = 128

NC = 2
NS = 16
NW = NC * NS
CH = 112
LANES = 16
N_PAD = 10240
ROWS_PER_TILE = N_PAD // NS
T_EDGES = -(-N_EDGES // (NW * CH)) * CH
E_PAD = T_EDGES * NW
N_CHUNKS = T_EDGES // CH
N_STAGES = N_CHUNKS + 2
T_ALLOC_CHUNKS = N_STAGES + 2
T_ALLOC = T_ALLOC_CHUNKS * CH


def _mm_body(f_ref, w_ref, o_ref):
    o_ref[...] = jnp.dot(f_ref[...], w_ref[...],
                         preferred_element_type=jnp.float32)


def _matmul(feat, W):
    m = feat.shape[0]
    bm = 1000
    return pl.pallas_call(
        _mm_body,
        grid=(m // bm,),
        in_specs=[
            pl.BlockSpec((bm, IN_FEAT), lambda i: (i, 0)),
            pl.BlockSpec((IN_FEAT, OUT_FEAT), lambda i: (0, 0)),
        ],
        out_specs=pl.BlockSpec((bm, OUT_FEAT), lambda i: (i, 0)),
        out_shape=jax.ShapeDtypeStruct((m, OUT_FEAT), jnp.float32),
    )(feat, W)


def _add_body(p_ref, o_ref):
    o_ref[...] = p_ref[0] + p_ref[1]


def _add_partials(partials):
    n = N_NODES
    bm = 1000
    return pl.pallas_call(
        _add_body,
        grid=(n // bm,),
        in_specs=[pl.BlockSpec((NC, bm, OUT_FEAT), lambda i: (0, i, 0))],
        out_specs=pl.BlockSpec((bm, OUT_FEAT), lambda i: (i, 0)),
        out_shape=jax.ShapeDtypeStruct((n, OUT_FEAT), jnp.float32),
    )(partials)


def _spmm_body(emb_hbm, src_hbm, dst_hbm, val_hbm, zeros_hbm, out_hbm,
               src0, src1, src2, dst0, dst1, dst2, val0, val1, val2,
               rows0, rows1, rows2, acc,
               gsem0, gsem1, gsem2, ssem0, ssem1, ssem2,
               isem0, isem1, isem2, dsem0, dsem1, dsem2):
    c = lax.axis_index("c")
    s = lax.axis_index("s")
    base_rows = s * ROWS_PER_TILE
    tile_base = (c * NS + s) * T_ALLOC

    pltpu.sync_copy(zeros_hbm, acc.at[pl.ds(base_rows, ROWS_PER_TILE)])
    plsc.subcore_barrier()

    srcs = (src0, src1, src2)
    dsts = (dst0, dst1, dst2)
    valsb = (val0, val1, val2)
    rows = (rows0, rows1, rows2)
    gsems = (gsem0, gsem1, gsem2)
    ssems = (ssem0, ssem1, ssem2)
    isems = (isem0, isem1, isem2)
    dsems = (dsem0, dsem1, dsem2)

    def sv_start(ch, r):
        base = tile_base + ch * CH
        pltpu.async_copy(src_hbm.at[pl.ds(base, CH)], srcs[r], isems[r])
        pltpu.async_copy(val_hbm.at[pl.ds(base, CH)], valsb[r], isems[r])

    def sv_wait(ch, r):
        base = tile_base + ch * CH
        pltpu.make_async_copy(src_hbm.at[pl.ds(base, CH)], srcs[r],
                              isems[r]).wait()
        pltpu.make_async_copy(val_hbm.at[pl.ds(base, CH)], valsb[r],
                              isems[r]).wait()

    def d_start(ch, r):
        base = tile_base + ch * CH
        pltpu.async_copy(dst_hbm.at[pl.ds(base, CH)], dsts[r], dsems[r])

    def d_wait(ch, r):
        base = tile_base + ch * CH
        pltpu.make_async_copy(dst_hbm.at[pl.ds(base, CH)], dsts[r],
                              dsems[r]).wait()

    def g_start(r):
        pltpu.async_copy(emb_hbm.at[srcs[r]], rows[r], gsems[r])

    def g_wait(r):
        pltpu.make_async_copy(emb_hbm.at[srcs[r]], rows[r], gsems[r]).wait()

    def sc_start(r):
        pltpu.async_copy(rows[r], acc.at[dsts[r]], ssems[r], add=True)

    def sc_wait(r):
        pltpu.make_async_copy(rows[r], acc.at[dsts[r]], ssems[r]).wait()

    def scale(r):
        buf = rows[r]
        valb = valsb[r]

        @pl.loop(0, CH // LANES)
        def _scale(g):
            vvec = valb[pl.ds(g * LANES, LANES)]
            for j in range(LANES):
                v = vvec[j]
                e = g * LANES + j
                for d in range(OUT_FEAT // LANES):
                    sl = pl.ds(d * LANES, LANES)
                    buf[e, sl] = buf[e, sl] * v

    sv_start(0, 0)
    sv_start(1, 1)
    d_start(0, 0)
    d_start(1, 1)
    sv_wait(0, 0)
    g_start(0)
    sv_wait(1, 1)
    g_start(1)
    g_wait(0)
    scale(0)
    d_wait(0, 0)
    sc_start(0)
    sv_start(2, 2)
    d_start(2, 2)
    sv_wait(2, 2)
    g_start(2)
    g_wait(1)
    scale(1)
    d_wait(1, 1)
    sc_start(1)
    sv_start(3, 0)

    @pl.loop(2, N_STAGES, step=3)
    def _stage(i):
        for b in range(3):
            j = i + b
            r = (2 + b) % 3
            rn = (r + 1) % 3
            rp = (r + 2) % 3
            sc_wait(rn)
            d_start(j + 1, rn)
            sv_wait(j + 1, rn)
            g_start(rn)
            g_wait(r)
            scale(r)
            d_wait(j, r)
            sc_start(r)
            sv_start(j + 2, rp)

    sc_wait(0)
    sc_wait(1)
    g_wait(2)
    sv_wait(N_STAGES + 1, 0)
    d_wait(N_STAGES, 2)

    plsc.subcore_barrier()
    pltpu.sync_copy(acc.at[pl.ds(base_rows, ROWS_PER_TILE)],
                    out_hbm.at[c, pl.ds(base_rows, ROWS_PER_TILE)])


_sc_mesh = plsc.VectorSubcoreMesh(core_axis_name="c", subcore_axis_name="s")

_spmm = pl.kernel(
    _spmm_body,
    out_type=jax.ShapeDtypeStruct((NC, N_PAD, OUT_FEAT), jnp.float32),
    mesh=_sc_mesh,
    scratch_types=[
        pltpu.VMEM((CH,), jnp.int32),
        pltpu.VMEM((CH,), jnp.int32),
        pltpu.VMEM((CH,), jnp.int32),
        pltpu.VMEM((CH,), jnp.int32),
        pltpu.VMEM((CH,), jnp.int32),
        pltpu.VMEM((CH,), jnp.int32),
        pltpu.VMEM((CH,), jnp.float32),
        pltpu.VMEM((CH,), jnp.float32),
        pltpu.VMEM((CH,), jnp.float32),
        pltpu.VMEM((CH, OUT_FEAT), jnp.float32),
        pltpu.VMEM((CH, OUT_FEAT), jnp.float32),
        pltpu.VMEM((CH, OUT_FEAT), jnp.float32),
        pltpu.VMEM_SHARED((N_PAD, OUT_FEAT), jnp.float32),
        pltpu.SemaphoreType.DMA,
        pltpu.SemaphoreType.DMA,
        pltpu.SemaphoreType.DMA,
        pltpu.SemaphoreType.DMA,
        pltpu.SemaphoreType.DMA,
        pltpu.SemaphoreType.DMA,
        pltpu.SemaphoreType.DMA,
        pltpu.SemaphoreType.DMA,
        pltpu.SemaphoreType.DMA,
        pltpu.SemaphoreType.DMA,
        pltpu.SemaphoreType.DMA,
        pltpu.SemaphoreType.DMA,
    ],
)


def kernel(feat, edge_index, adj_values, W):
    emb = _matmul(feat, W)

    src = edge_index[1].astype(jnp.int32)
    dst = edge_index[0].astype(jnp.int32)
    vals = adj_values.astype(jnp.float32)
    pad = E_PAD - N_EDGES

    def pad_tiles(a):
        a = jnp.concatenate([a, jnp.zeros((pad,), a.dtype)])
        a = a.reshape(NW, T_EDGES)
        extra = T_ALLOC - T_EDGES
        a = jnp.concatenate([a, jnp.zeros((NW, extra), a.dtype)], axis=1)
        return a.reshape(NW * T_ALLOC)

    src = pad_tiles(src)
    dst = pad_tiles(dst)
    vals = pad_tiles(vals)
    zeros = jnp.zeros((ROWS_PER_TILE, OUT_FEAT), jnp.float32)

    partials = _spmm(emb, src, dst, vals, zeros)
    x = _add_partials(partials)
    return (emb, x)


# speedup vs baseline: 1.5991x; 1.5991x over previous
import jax
import jax.numpy as jnp
from jax import lax
from jax.experimental import pallas as pl
from jax.experimental.pallas import tpu as pltpu
from jax.experimental.pallas import tpu_sc as plsc

N_NODES = 10000
N_EDGES = 320000
IN_FEAT = 128
OUT_FEAT = 128

NC = 2
NS = 16
NW = NC * NS
CH = 128
LANES = 16
N_PAD = 10240
ROWS_PER_TILE = N_PAD // NS
T_EDGES = -(-N_EDGES // (NW * CH)) * CH
E_PAD = T_EDGES * NW
N_CHUNKS = T_EDGES // CH


def _mm_body(f_ref, w_ref, o_ref):
    o_ref[...] = jnp.dot(f_ref[...], w_ref[...],
                         preferred_element_type=jnp.float32)


def _matmul(feat, W):
    m = feat.shape[0]
    bm = 1000
    return pl.pallas_call(
        _mm_body,
        grid=(m // bm,),
        in_specs=[
            pl.BlockSpec((bm, IN_FEAT), lambda i: (i, 0)),
            pl.BlockSpec((IN_FEAT, OUT_FEAT), lambda i: (0, 0)),
        ],
        out_specs=pl.BlockSpec((bm, OUT_FEAT), lambda i: (i, 0)),
        out_shape=jax.ShapeDtypeStruct((m, OUT_FEAT), jnp.float32),
    )(feat, W)


def _add_body(p_ref, o_ref):
    o_ref[...] = p_ref[0] + p_ref[1]


def _add_partials(partials):
    n = N_NODES
    bm = 1000
    return pl.pallas_call(
        _add_body,
        grid=(n // bm,),
        in_specs=[pl.BlockSpec((NC, bm, OUT_FEAT), lambda i: (0, i, 0))],
        out_specs=pl.BlockSpec((bm, OUT_FEAT), lambda i: (i, 0)),
        out_shape=jax.ShapeDtypeStruct((n, OUT_FEAT), jnp.float32),
    )(partials)


def _spmm_body(emb_hbm, src_hbm, dst_hbm, val_hbm, zeros_hbm, out_hbm,
               src_a, dst_a, val_a, srcc, dstc, rowsb, acc, gsem):
    c = lax.axis_index("c")
    s = lax.axis_index("s")
    wid = c * NS + s
    base_rows = s * ROWS_PER_TILE

    pltpu.sync_copy(zeros_hbm, acc.at[pl.ds(base_rows, ROWS_PER_TILE)])
    pltpu.sync_copy(src_hbm.at[wid], src_a)
    pltpu.sync_copy(dst_hbm.at[wid], dst_a)
    pltpu.sync_copy(val_hbm.at[wid], val_a)
    plsc.subcore_barrier()

    @pl.loop(0, N_CHUNKS)
    def _chunk(i):
        for g in range(CH // LANES):
            sl = pl.ds(g * LANES, LANES)
            srcc[sl] = src_a[i, sl]

        gather = pltpu.async_copy(emb_hbm.at[srcc], rowsb, gsem)
        for g in range(CH // LANES):
            sl = pl.ds(g * LANES, LANES)
            dstc[sl] = dst_a[i, sl]
        gather.wait()

        @pl.loop(0, CH // LANES)
        def _scale(g):
            vvec = val_a[i, pl.ds(g * LANES, LANES)]
            for j in range(LANES):
                v = vvec[j]
                e = g * LANES + j
                for d in range(OUT_FEAT // LANES):
                    sl = pl.ds(d * LANES, LANES)
                    rowsb[e, sl] = rowsb[e, sl] * v

        pltpu.sync_copy(rowsb, acc.at[dstc], add=True)

    plsc.subcore_barrier()
    pltpu.sync_copy(acc.at[pl.ds(base_rows, ROWS_PER_TILE)],
                    out_hbm.at[c, pl.ds(base_rows, ROWS_PER_TILE)])


_sc_mesh = plsc.VectorSubcoreMesh(core_axis_name="c", subcore_axis_name="s")

_spmm = pl.kernel(
    _spmm_body,
    out_type=jax.ShapeDtypeStruct((NC, N_PAD, OUT_FEAT), jnp.float32),
    mesh=_sc_mesh,
    scratch_types=[
        pltpu.VMEM((N_CHUNKS, CH), jnp.int32),
        pltpu.VMEM((N_CHUNKS, CH), jnp.int32),
        pltpu.VMEM((N_CHUNKS, CH), jnp.float32),
        pltpu.VMEM((CH,), jnp.int32),
        pltpu.VMEM((CH,), jnp.int32),
        pltpu.VMEM((CH, OUT_FEAT), jnp.float32),
        pltpu.VMEM_SHARED((N_PAD, OUT_FEAT), jnp.float32),
        pltpu.SemaphoreType.DMA,
    ],
)


def kernel(feat, edge_index, adj_values, W):
    emb = _matmul(feat, W)

    src = edge_index[1].astype(jnp.int32)
    dst = edge_index[0].astype(jnp.int32)
    vals = adj_values.astype(jnp.float32)
    pad = E_PAD - N_EDGES
    src = jnp.concatenate([src, jnp.zeros((pad,), jnp.int32)])
    dst = jnp.concatenate([dst, jnp.zeros((pad,), jnp.int32)])
    vals = jnp.concatenate([vals, jnp.zeros((pad,), jnp.float32)])
    src = src.reshape(NW, N_CHUNKS, CH)
    dst = dst.reshape(NW, N_CHUNKS, CH)
    vals = vals.reshape(NW, N_CHUNKS, CH)
    zeros = jnp.zeros((ROWS_PER_TILE, OUT_FEAT), jnp.float32)

    partials = _spmm(emb, src, dst, vals, zeros)
    x = _add_partials(partials)
    return (emb, x)


# speedup vs baseline: 1.6333x; 1.0214x over previous
import jax
import jax.numpy as jnp
from jax import lax
from jax.experimental import pallas as pl
from jax.experimental.pallas import tpu as pltpu
from jax.experimental.pallas import tpu_sc as plsc

N_NODES = 10000
N_EDGES = 320000
IN_FEAT = 128
OUT_FEAT = 128

NC = 2
NS = 16
NW = NC * NS
CH = 128
LANES = 16
N_PAD = 10240
ROWS_PER_TILE = N_PAD // NS
T_EDGES = -(-N_EDGES // (NW * CH)) * CH
E_PAD = T_EDGES * NW
N_CHUNKS = T_EDGES // CH


def _mm_body(f_ref, w_ref, o_ref):
    o_ref[...] = jnp.dot(f_ref[...], w_ref[...],
                         preferred_element_type=jnp.float32)


def _matmul(feat, W):
    m = feat.shape[0]
    bm = 1000
    return pl.pallas_call(
        _mm_body,
        grid=(m // bm,),
        in_specs=[
            pl.BlockSpec((bm, IN_FEAT), lambda i: (i, 0)),
            pl.BlockSpec((IN_FEAT, OUT_FEAT), lambda i: (0, 0)),
        ],
        out_specs=pl.BlockSpec((bm, OUT_FEAT), lambda i: (i, 0)),
        out_shape=jax.ShapeDtypeStruct((m, OUT_FEAT), jnp.float32),
    )(feat, W)


def _add_body(p_ref, o_ref):
    o_ref[...] = p_ref[0] + p_ref[1]


def _add_partials(partials):
    n = N_NODES
    bm = 1000
    return pl.pallas_call(
        _add_body,
        grid=(n // bm,),
        in_specs=[pl.BlockSpec((NC, bm, OUT_FEAT), lambda i: (0, i, 0))],
        out_specs=pl.BlockSpec((bm, OUT_FEAT), lambda i: (i, 0)),
        out_shape=jax.ShapeDtypeStruct((n, OUT_FEAT), jnp.float32),
    )(partials)


HALF = 64
N_HALF = T_EDGES // HALF
N_HROWS = (N_HALF + 2) // 2


def _spmm_body(emb_hbm, src_hbm, dst_hbm, val_hbm, zeros_hbm, out_hbm,
               src_a, dst_a, val_a, srcc0, srcc1, dstc0, dstc1,
               rows0, rows1, acc, gsem0, gsem1):
    c = lax.axis_index("c")
    s = lax.axis_index("s")
    wid = c * NS + s
    base_rows = s * ROWS_PER_TILE

    pltpu.sync_copy(zeros_hbm, acc.at[pl.ds(base_rows, ROWS_PER_TILE)])
    pltpu.sync_copy(src_hbm.at[wid], src_a)
    pltpu.sync_copy(dst_hbm.at[wid], dst_a)
    pltpu.sync_copy(val_hbm.at[wid], val_a)
    plsc.subcore_barrier()

    rows = (rows0, rows1)
    srcs = (srcc0, srcc1)
    dsts = (dstc0, dstc1)
    gsems = (gsem0, gsem1)

    def copy_idx(staged, dest, k):
        row = k // 2
        off = (k % 2) * HALF
        for g in range(HALF // LANES):
            dest[pl.ds(g * LANES, LANES)] = (
                staged[row, pl.ds(off + g * LANES, LANES)])

    def g_start(b):
        pltpu.async_copy(emb_hbm.at[srcs[b]], rows[b], gsems[b])

    def g_wait(b):
        pltpu.make_async_copy(emb_hbm.at[srcs[b]], rows[b], gsems[b]).wait()

    copy_idx(src_a, srcc0, 0)
    g_start(0)

    @pl.loop(0, N_HALF, step=2)
    def _chunk(i):
        for b in range(2):
            k = i + b
            nb = 1 - b
            copy_idx(src_a, srcs[nb], k + 1)
            g_start(nb)

            g_wait(b)

            buf = rows[b]
            row = k // 2
            off = (k % 2) * HALF

            @pl.loop(0, HALF // LANES)
            def _scale(g):
                vvec = val_a[row, pl.ds(off + g * LANES, LANES)]
                for j in range(LANES):
                    v = vvec[j]
                    e = g * LANES + j
                    for d in range(OUT_FEAT // LANES):
                        sl = pl.ds(d * LANES, LANES)
                        buf[e, sl] = buf[e, sl] * v

            copy_idx(dst_a, dsts[b], k)
            pltpu.sync_copy(buf, acc.at[dsts[b]], add=True)

    g_wait(0)

    plsc.subcore_barrier()
    pltpu.sync_copy(acc.at[pl.ds(base_rows, ROWS_PER_TILE)],
                    out_hbm.at[c, pl.ds(base_rows, ROWS_PER_TILE)])


_sc_mesh = plsc.VectorSubcoreMesh(core_axis_name="c", subcore_axis_name="s")

_spmm = pl.kernel(
    _spmm_body,
    out_type=jax.ShapeDtypeStruct((NC, N_PAD, OUT_FEAT), jnp.float32),
    mesh=_sc_mesh,
    scratch_types=[
        pltpu.VMEM((N_HROWS, CH), jnp.int32),
        pltpu.VMEM((N_HROWS, CH), jnp.int32),
        pltpu.VMEM((N_HROWS, CH), jnp.float32),
        pltpu.VMEM((HALF,), jnp.int32),
        pltpu.VMEM((HALF,), jnp.int32),
        pltpu.VMEM((HALF,), jnp.int32),
        pltpu.VMEM((HALF,), jnp.int32),
        pltpu.VMEM((HALF, OUT_FEAT), jnp.float32),
        pltpu.VMEM((HALF, OUT_FEAT), jnp.float32),
        pltpu.VMEM_SHARED((N_PAD, OUT_FEAT), jnp.float32),
        pltpu.SemaphoreType.DMA,
        pltpu.SemaphoreType.DMA,
    ],
)


def kernel(feat, edge_index, adj_values, W):
    emb = _matmul(feat, W)

    src = edge_index[1].astype(jnp.int32)
    dst = edge_index[0].astype(jnp.int32)
    vals = adj_values.astype(jnp.float32)
    pad = E_PAD - N_EDGES

    def pad_tiles(a):
        a = jnp.concatenate([a, jnp.zeros((pad,), a.dtype)])
        a = a.reshape(NW, T_EDGES)
        extra = N_HROWS * CH - T_EDGES
        a = jnp.concatenate([a, jnp.zeros((NW, extra), a.dtype)], axis=1)
        return a.reshape(NW, N_HROWS, CH)

    src = pad_tiles(src)
    dst = pad_tiles(dst)
    vals = pad_tiles(vals)
    zeros = jnp.zeros((ROWS_PER_TILE, OUT_FEAT), jnp.float32)

    partials = _spmm(emb, src, dst, vals, zeros)
    x = _add_partials(partials)
    return (emb, x)
